# baseline (speedup 1.0000x reference)
import functools

import jax
import jax.numpy as jnp
from jax import lax
from jax.experimental import pallas as pl
from jax.experimental.pallas import tpu as pltpu
from jax.experimental.pallas import tpu_sc as plsc

N_NODES = 10000
N_EDGES = 320000
ND = 16
NC, NS, L = 2, 16, 16
PK = 8
EH = N_EDGES // 2
EP2 = EH // PK

CHG = 40
EPW = EH // (NC * NS)
JG = EPW // CHG

CHS = 80
EPS = EH // NS
JS = EPS // CHS
HALF = N_NODES // NC
ACC_ROWS = HALF + 8
ORO = HALF // NS
OREM = HALF - NS * ORO
RSL = 78

@functools.cache
def _make_gather_sc():
    mesh = plsc.VectorSubcoreMesh(
        core_axis_name="c", subcore_axis_name="s", num_cores=NC, num_subcores=NS
    )
    return pl.kernel(
        _gather_body,
        out_type=jax.ShapeDtypeStruct((EH, ND), jnp.float32),
        mesh=mesh,
        scratch_types=[
            pltpu.VMEM((JG, CHG), jnp.int32),
            pltpu.VMEM((CHG, ND), jnp.float32),
            pltpu.VMEM((CHG, ND), jnp.float32),
            pltpu.SemaphoreType.DMA,
            pltpu.SemaphoreType.DMA,
        ],
        compiler_params=pltpu.CompilerParams(use_tc_tiling_on_sc=False),
    )


def _gather_body(node_hbm, src3d_hbm, nbr_hbm, idx_v, row0_v, row1_v, sem0, sem1):
    wid = lax.axis_index("s") * NC + lax.axis_index("c")
    pltpu.sync_copy(src3d_hbm.at[wid], idx_v)
    obase = wid * EPW

    pltpu.async_copy(node_hbm.at[idx_v.at[0]], row0_v, sem0)

    @pl.loop(0, JG // 2)
    def _(jj):
        j0 = jj * 2
        pltpu.make_async_copy(node_hbm.at[idx_v.at[j0]], row0_v, sem0).wait()
        pltpu.async_copy(node_hbm.at[idx_v.at[j0 + 1]], row1_v, sem1)
        pltpu.sync_copy(row0_v, nbr_hbm.at[pl.ds(obase + j0 * CHG, CHG)])
        nxt = jnp.where(j0 + 2 < JG, j0 + 2, 0)
        pltpu.async_copy(node_hbm.at[idx_v.at[nxt]], row0_v, sem0)
        pltpu.make_async_copy(node_hbm.at[idx_v.at[j0 + 1]], row1_v, sem1).wait()
        pltpu.sync_copy(row1_v, nbr_hbm.at[pl.ds(obase + (j0 + 1) * CHG, CHG)])

    j_last = JG - 1
    pltpu.make_async_copy(node_hbm.at[idx_v.at[j_last]], row0_v, sem0).wait()
    pltpu.sync_copy(row0_v, nbr_hbm.at[pl.ds(obase + j_last * CHG, CHG)])


RB = 800
NBLK = EP2 // RB


def _tc_body(ef_ref, nbr_ref, w8p_ref, b8p_ref, r8_ref, out_ref):
    ef8 = ef_ref[...]
    nbr8 = nbr_ref[...]
    w8p = w8p_ref[...]
    b8p = b8p_ref[...]
    r8 = r8_ref[...]
    efw = jnp.dot(ef8, w8p, preferred_element_type=jnp.float32) + b8p
    nrep = jnp.dot(nbr8, r8, preferred_element_type=jnp.float32)
    prod = efw * nrep
    pw = PK * ND
    acc = prod[:, 0:pw]
    for j in range(1, ND):
        acc = acc + prod[:, j * pw:(j + 1) * pw]
    out_ref[...] = acc


_transform_tc = pl.pallas_call(
    _tc_body,
    grid=(NBLK,),
    in_specs=[
        pl.BlockSpec((RB, PK * ND), lambda i: (i, 0)),
        pl.BlockSpec((RB, PK * ND), lambda i: (i, 0)),
        pl.BlockSpec((PK * ND, PK * ND * ND), lambda i: (0, 0)),
        pl.BlockSpec((1, PK * ND * ND), lambda i: (0, 0)),
        pl.BlockSpec((PK * ND, PK * ND * ND), lambda i: (0, 0)),
    ],
    out_specs=pl.BlockSpec((RB, PK * ND), lambda i: (i, 0)),
    out_shape=jax.ShapeDtypeStruct((EP2, PK * ND), jnp.float32),
)


@functools.cache
def _make_scatter_sc():
    mesh = plsc.VectorSubcoreMesh(
        core_axis_name="c", subcore_axis_name="s", num_cores=NC, num_subcores=NS
    )
    return pl.kernel(
        _scatter_body,
        out_type=jax.ShapeDtypeStruct((N_NODES, ND), jnp.float32),
        mesh=mesh,
        scratch_types=[
            pltpu.VMEM((JS, CHS), jnp.int32),
            pltpu.VMEM((CHS, ND), jnp.float32),
            pltpu.VMEM((CHS, ND), jnp.float32),
            pltpu.VMEM((RSL, ND), jnp.float32),
            pltpu.VMEM((NS * RSL, ND), jnp.float32),
            pltpu.SemaphoreType.DMA,
            pltpu.SemaphoreType.DMA,
            pltpu.VMEM_SHARED((NS * ACC_ROWS, ND), jnp.float32),
        ],
        compiler_params=pltpu.CompilerParams(use_tc_tiling_on_sc=False),
    )


def _reduce_rows(acc, red16_v, red_v, out_hbm, start, nrows, base):
    for k in range(NS):
        pltpu.sync_copy(
            acc.at[pl.ds(k * ACC_ROWS + start, nrows)],
            red16_v.at[pl.ds(k * nrows, nrows)],
        )

    @pl.loop(0, nrows)
    def _(r):
        s = red16_v[r]
        for k in range(1, NS):
            s = s + red16_v[k * nrows + r]
        red_v[r] = s

    pltpu.sync_copy(
        red_v.at[pl.ds(0, nrows)], out_hbm.at[pl.ds(base + start, nrows)]
    )


def _scatter_body(
    t_hbm, dst3d_hbm, zeros_hbm, init_hbm, out_hbm,
    idx_v, row0_v, row1_v, red_v, red16_v, sem0, sem1, acc,
):
    cid = lax.axis_index("c")
    sid = lax.axis_index("s")
    base = cid * HALF
    off = sid * ACC_ROWS
    ebase = sid * EPS

    def t_chunk(j):
        return t_hbm.at[pl.ds(ebase + j * CHS, CHS)]

    pltpu.async_copy(t_chunk(0), row0_v, sem0)

    @pl.when(sid == 0)
    def _():
        pltpu.sync_copy(init_hbm.at[pl.ds(base, HALF)], acc.at[pl.ds(off, HALF)])
        pltpu.sync_copy(
            zeros_hbm.at[pl.ds(0, ACC_ROWS - HALF)],
            acc.at[pl.ds(off + HALF, ACC_ROWS - HALF)],
        )

    @pl.when(sid != 0)
    def _():
        pltpu.sync_copy(zeros_hbm.at[pl.ds(0, ACC_ROWS)], acc.at[pl.ds(off, ACC_ROWS)])

    pltpu.sync_copy(dst3d_hbm.at[sid], idx_v)

    @pl.loop(0, JS)
    def _(j):
        for k in range(CHS // L):
            v = idx_v[j, k * L:(k + 1) * L] - base
            ok = (v >= 0) & (v < HALF)
            idx_v[j, k * L:(k + 1) * L] = off + jnp.where(ok, v, HALF)

    @pl.loop(0, JS // 2)
    def _(jj):
        j0 = jj * 2
        pltpu.make_async_copy(t_chunk(j0), row0_v, sem0).wait()
        pltpu.async_copy(t_chunk(j0 + 1), row1_v, sem1)
        pltpu.sync_copy(row0_v, acc.at[idx_v.at[j0]], add=True)
        nxt = jnp.where(j0 + 2 < JS, j0 + 2, 0)
        pltpu.async_copy(t_chunk(nxt), row0_v, sem0)
        pltpu.make_async_copy(t_chunk(j0 + 1), row1_v, sem1).wait()
        pltpu.sync_copy(row1_v, acc.at[idx_v.at[j0 + 1]], add=True)

    pltpu.make_async_copy(t_chunk(JS - 1), row0_v, sem0).wait()
    pltpu.sync_copy(row0_v, acc.at[idx_v.at[JS - 1]], add=True)

    plsc.subcore_barrier()

    for t in range(ORO // RSL):
        _reduce_rows(acc, red16_v, red_v, out_hbm, sid * ORO + t * RSL, RSL, base)

    @pl.when(sid == NS - 1)
    def _():
        _reduce_rows(acc, red16_v, red_v, out_hbm, NS * ORO, OREM, base)


def kernel(node_features, edge_features, pair_indices, kernel, bias):
    src = pair_indices[:, 1]
    dst = pair_indices[:, 0]
    srcA = src[:EH].reshape(NC * NS, JG, CHG)
    srcB = src[EH:].reshape(NC * NS, JG, CHG)
    dstA = dst[:EH].reshape(NS, JS, CHS)
    dstB = dst[EH:].reshape(NS, JS, CHS)

    eye8 = jnp.eye(PK, dtype=jnp.float32)
    w8p = jnp.concatenate(
        [jnp.kron(eye8, kernel[:, j::ND]) for j in range(ND)], axis=1
    )
    b8p = jnp.concatenate(
        [jnp.tile(bias[j::ND], PK) for j in range(ND)]
    ).reshape(1, PK * ND * ND)
    r8 = jnp.concatenate(
        [
            jnp.kron(eye8, jnp.zeros((ND, ND), jnp.float32).at[j].set(1.0))
            for j in range(ND)
        ],
        axis=1,
    )
    ef8 = edge_features.reshape(2 * EP2, PK * ND)
    efA8 = ef8[:EP2]
    efB8 = ef8[EP2:]

    gather = _make_gather_sc()
    scatter = _make_scatter_sc()
    zeros_acc = jnp.zeros((ACC_ROWS, ND), jnp.float32)
    zeros_n = jnp.zeros((N_NODES, ND), jnp.float32)

    nbrA = gather(node_features, srcA)
    nbrB = gather(node_features, srcB)
    tA = _transform_tc(efA8, nbrA.reshape(EP2, PK * ND), w8p, b8p, r8)
    tB = _transform_tc(efB8, nbrB.reshape(EP2, PK * ND), w8p, b8p, r8)
    outA = scatter(tA.reshape(EH, ND), dstA, zeros_acc, zeros_n)
    return scatter(tB.reshape(EH, ND), dstB, zeros_acc, outA)


# speedup vs baseline: 1.2295x; 1.2295x over previous
import functools

import jax
import jax.numpy as jnp
from jax import lax
from jax.experimental import pallas as pl
from jax.experimental.pallas import tpu as pltpu
from jax.experimental.pallas import tpu_sc as plsc

N_NODES = 10000
N_EDGES = 320000
ND = 16
NC, NS, L = 2, 16, 16
PK = 8
EP = N_EDGES // PK

CHUNK = 80
EPW = N_EDGES // (NC * NS)
JG = EPW // CHUNK

EPS = N_EDGES // NS
JS = EPS // CHUNK
HALF = N_NODES // NC
ACC_ROWS = HALF + 8
ORO = HALF // NS
OREM = HALF - NS * ORO
RSL = 78

@functools.cache
def _make_gather_sc():
    mesh = plsc.VectorSubcoreMesh(
        core_axis_name="c", subcore_axis_name="s", num_cores=NC, num_subcores=NS
    )
    return pl.kernel(
        _gather_body,
        out_type=jax.ShapeDtypeStruct((N_EDGES, ND), jnp.float32),
        mesh=mesh,
        scratch_types=[
            pltpu.VMEM((JG, CHUNK), jnp.int32),
            pltpu.VMEM((CHUNK, ND), jnp.float32),
            pltpu.VMEM((CHUNK, ND), jnp.float32),
            pltpu.SemaphoreType.DMA,
            pltpu.SemaphoreType.DMA,
        ],
        compiler_params=pltpu.CompilerParams(use_tc_tiling_on_sc=False),
    )


def _gather_body(node_hbm, src3d_hbm, nbr_hbm, idx_v, row0_v, row1_v, sem0, sem1):
    wid = lax.axis_index("s") * NC + lax.axis_index("c")
    pltpu.sync_copy(src3d_hbm.at[wid], idx_v)
    obase = wid * EPW

    pltpu.async_copy(node_hbm.at[idx_v.at[0]], row0_v, sem0)

    @pl.loop(0, JG // 2)
    def _(jj):
        j0 = jj * 2
        pltpu.make_async_copy(node_hbm.at[idx_v.at[j0]], row0_v, sem0).wait()
        pltpu.async_copy(node_hbm.at[idx_v.at[j0 + 1]], row1_v, sem1)
        pltpu.sync_copy(row0_v, nbr_hbm.at[pl.ds(obase + j0 * CHUNK, CHUNK)])
        nxt = jnp.where(j0 + 2 < JG, j0 + 2, 0)
        pltpu.async_copy(node_hbm.at[idx_v.at[nxt]], row0_v, sem0)
        pltpu.make_async_copy(node_hbm.at[idx_v.at[j0 + 1]], row1_v, sem1).wait()
        pltpu.sync_copy(row1_v, nbr_hbm.at[pl.ds(obase + (j0 + 1) * CHUNK, CHUNK)])

    j_last = JG - 1
    pltpu.make_async_copy(node_hbm.at[idx_v.at[j_last]], row0_v, sem0).wait()
    pltpu.sync_copy(row0_v, nbr_hbm.at[pl.ds(obase + j_last * CHUNK, CHUNK)])


RB = 800
NBLK = EP // RB


def _tc_body(ef_ref, nbr_ref, w8p_ref, b8p_ref, r8_ref, out_ref):
    ef8 = ef_ref[...]
    nbr8 = nbr_ref[...]
    w8p = w8p_ref[...]
    b8p = b8p_ref[...]
    r8 = r8_ref[...]
    efw = jnp.dot(ef8, w8p, preferred_element_type=jnp.float32) + b8p
    nrep = jnp.dot(nbr8, r8, preferred_element_type=jnp.float32)
    prod = efw * nrep
    pw = PK * ND
    acc = prod[:, 0:pw]
    for j in range(1, ND):
        acc = acc + prod[:, j * pw:(j + 1) * pw]
    out_ref[...] = acc


_transform_tc = pl.pallas_call(
    _tc_body,
    grid=(NBLK,),
    in_specs=[
        pl.BlockSpec((RB, PK * ND), lambda i: (i, 0)),
        pl.BlockSpec((RB, PK * ND), lambda i: (i, 0)),
        pl.BlockSpec((PK * ND, PK * ND * ND), lambda i: (0, 0)),
        pl.BlockSpec((1, PK * ND * ND), lambda i: (0, 0)),
        pl.BlockSpec((PK * ND, PK * ND * ND), lambda i: (0, 0)),
    ],
    out_specs=pl.BlockSpec((RB, PK * ND), lambda i: (i, 0)),
    out_shape=jax.ShapeDtypeStruct((EP, PK * ND), jnp.float32),
)


@functools.cache
def _make_scatter_sc():
    mesh = plsc.VectorSubcoreMesh(
        core_axis_name="c", subcore_axis_name="s", num_cores=NC, num_subcores=NS
    )
    return pl.kernel(
        _scatter_body,
        out_type=jax.ShapeDtypeStruct((N_NODES, ND), jnp.float32),
        mesh=mesh,
        scratch_types=[
            pltpu.VMEM((JS, CHUNK), jnp.int32),
            pltpu.VMEM((CHUNK, ND), jnp.float32),
            pltpu.VMEM((CHUNK, ND), jnp.float32),
            pltpu.VMEM((RSL, ND), jnp.float32),
            pltpu.VMEM((NS * RSL, ND), jnp.float32),
            pltpu.SemaphoreType.DMA,
            pltpu.SemaphoreType.DMA,
            pltpu.VMEM_SHARED((NS * ACC_ROWS, ND), jnp.float32),
        ],
        compiler_params=pltpu.CompilerParams(use_tc_tiling_on_sc=False),
    )


def _reduce_rows(acc, red16_v, red_v, out_hbm, start, nrows, base):
    for k in range(NS):
        pltpu.sync_copy(
            acc.at[pl.ds(k * ACC_ROWS + start, nrows)],
            red16_v.at[pl.ds(k * nrows, nrows)],
        )

    @pl.loop(0, nrows)
    def _(r):
        s = red16_v[r]
        for k in range(1, NS):
            s = s + red16_v[k * nrows + r]
        red_v[r] = s

    pltpu.sync_copy(
        red_v.at[pl.ds(0, nrows)], out_hbm.at[pl.ds(base + start, nrows)]
    )


def _scatter_body(
    t_hbm, dst3d_hbm, zeros_hbm, out_hbm,
    idx_v, row0_v, row1_v, red_v, red16_v, sem0, sem1, acc,
):
    cid = lax.axis_index("c")
    sid = lax.axis_index("s")
    base = cid * HALF
    off = sid * ACC_ROWS
    ebase = sid * EPS

    def t_chunk(j):
        return t_hbm.at[pl.ds(ebase + j * CHUNK, CHUNK)]

    pltpu.async_copy(t_chunk(0), row0_v, sem0)

    pltpu.sync_copy(zeros_hbm.at[pl.ds(0, ACC_ROWS)], acc.at[pl.ds(off, ACC_ROWS)])

    pltpu.sync_copy(dst3d_hbm.at[sid], idx_v)

    @pl.loop(0, JS)
    def _(j):
        for k in range(CHUNK // L):
            v = idx_v[j, k * L:(k + 1) * L] - base
            ok = (v >= 0) & (v < HALF)
            idx_v[j, k * L:(k + 1) * L] = off + jnp.where(ok, v, HALF)

    @pl.loop(0, JS // 2 - 1)
    def _(jj):
        j0 = jj * 2
        pltpu.make_async_copy(t_chunk(j0), row0_v, sem0).wait()
        pltpu.async_copy(t_chunk(j0 + 1), row1_v, sem1)
        pltpu.sync_copy(row0_v, acc.at[idx_v.at[j0]], add=True)
        pltpu.async_copy(t_chunk(j0 + 2), row0_v, sem0)
        pltpu.make_async_copy(t_chunk(j0 + 1), row1_v, sem1).wait()
        pltpu.sync_copy(row1_v, acc.at[idx_v.at[j0 + 1]], add=True)

    pltpu.make_async_copy(t_chunk(JS - 2), row0_v, sem0).wait()
    pltpu.async_copy(t_chunk(JS - 1), row1_v, sem1)
    pltpu.sync_copy(row0_v, acc.at[idx_v.at[JS - 2]], add=True)
    pltpu.make_async_copy(t_chunk(JS - 1), row1_v, sem1).wait()
    pltpu.sync_copy(row1_v, acc.at[idx_v.at[JS - 1]], add=True)

    plsc.subcore_barrier()

    for t in range(ORO // RSL):
        _reduce_rows(acc, red16_v, red_v, out_hbm, sid * ORO + t * RSL, RSL, base)

    @pl.when(sid == NS - 1)
    def _():
        _reduce_rows(acc, red16_v, red_v, out_hbm, NS * ORO, OREM, base)


def kernel(node_features, edge_features, pair_indices, kernel, bias):
    src3d = pair_indices[:, 1].reshape(NC * NS, JG, CHUNK)
    dst3d = pair_indices[:, 0].reshape(NS, JS, CHUNK)

    eye8 = jnp.eye(PK, dtype=jnp.float32)
    w8p = jnp.concatenate(
        [jnp.kron(eye8, kernel[:, j::ND]) for j in range(ND)], axis=1
    )
    b8p = jnp.concatenate(
        [jnp.tile(bias[j::ND], PK) for j in range(ND)]
    ).reshape(1, PK * ND * ND)
    r8 = jnp.concatenate(
        [
            jnp.kron(eye8, jnp.zeros((ND, ND), jnp.float32).at[j].set(1.0))
            for j in range(ND)
        ],
        axis=1,
    )
    ef8 = edge_features.reshape(EP, PK * ND)

    nbr = _make_gather_sc()(node_features, src3d)
    nbr8 = nbr.reshape(EP, PK * ND)
    t8out = _transform_tc(ef8, nbr8, w8p, b8p, r8)
    zeros = jnp.zeros((ACC_ROWS, ND), jnp.float32)
    return _make_scatter_sc()(t8out.reshape(N_EDGES, ND), dst3d, zeros)
